# baseline (speedup 1.0000x reference)
import functools
import jax
import jax.numpy as jnp
from jax.experimental import pallas as pl
from jax.experimental.pallas import tpu as pltpu


def _pick(n, cands):
    for c in cands:
        if n % c == 0:
            return c
    return 1


def _mm_body(x_ref, w_ref, b_ref, o_ref, *, relu):
    acc = jnp.dot(x_ref[...], w_ref[...], preferred_element_type=jnp.float32)
    acc = acc + b_ref[...]
    if relu:
        acc = jnp.maximum(acc, 0.0)
    o_ref[...] = acc


def _matmul(x, w, b=None, relu=False):
    m, k = x.shape
    k2, n = w.shape
    if b is None:
        b = jnp.zeros((n,), jnp.float32)
    mb = _pick(m, [1000, 500, 250, 200, 125, 100, 8, 4, 2])
    grid = (m // mb,)
    return pl.pallas_call(
        functools.partial(_mm_body, relu=relu),
        grid=grid,
        in_specs=[
            pl.BlockSpec((mb, k), lambda i: (i, 0)),
            pl.BlockSpec((k2, n), lambda i: (0, 0)),
            pl.BlockSpec((1, n), lambda i: (0, 0)),
        ],
        out_specs=pl.BlockSpec((mb, n), lambda i: (i, 0)),
        out_shape=jax.ShapeDtypeStruct((m, n), jnp.float32),
    )(x, w, b.reshape(1, n))


def _hist_body(dst_ref, o_ref, *, ce):
    c = pl.program_id(0)

    @pl.when(c == 0)
    def _():
        o_ref[...] = jnp.zeros_like(o_ref)

    lanes = jax.lax.broadcasted_iota(jnp.int32, (1, 128), 1)

    def body(e, _):
        d = dst_ref[0, 0, e]
        r = d // 128
        l = d % 128
        o_ref[pl.ds(r, 1), :] += (lanes == l).astype(jnp.float32)
        return 0

    jax.lax.fori_loop(0, ce, body, 0)


def _degree(dst, n):
    e = dst.shape[0]
    ce = _pick(e, [4000, 2000, 1000, 500, 250, 125, 100, 50, 25, 10, 5, 2])
    nr = (n + 127) // 128
    nrp = ((nr + 7) // 8) * 8
    out = pl.pallas_call(
        functools.partial(_hist_body, ce=ce),
        grid=(e // ce,),
        in_specs=[
            pl.BlockSpec((1, 1, ce), lambda c: (c, 0, 0),
                         memory_space=pltpu.SMEM),
        ],
        out_specs=pl.BlockSpec((nrp, 128), lambda c: (0, 0)),
        out_shape=jax.ShapeDtypeStruct((nrp, 128), jnp.float32),
    )(dst.reshape(e // ce, 1, ce))
    return out.reshape(-1)[:n]


def _gather_body(idx_ref, h_ref, o_ref, *, ce):
    def body(e, _):
        s = idx_ref[0, 0, e]
        o_ref[pl.ds(e, 1), :] = h_ref[pl.ds(s, 1), :]
        return 0

    jax.lax.fori_loop(0, ce, body, 0)


def _scatter_body(dst_ref, m_ref, o_ref, *, ce):
    c = pl.program_id(0)

    @pl.when(c == 0)
    def _():
        o_ref[...] = jnp.zeros_like(o_ref)

    def body(e, _):
        d = dst_ref[0, 0, e]
        o_ref[pl.ds(d, 1), :] += m_ref[pl.ds(e, 1), :]
        return 0

    jax.lax.fori_loop(0, ce, body, 0)


def _scatter_half(src, dst, h):
    n, f = h.shape
    e = src.shape[0]
    ce = _pick(e, [4000, 2000, 1000, 500, 250, 125, 100, 50, 25, 10, 5, 2])
    src3 = src.reshape(e // ce, 1, ce)
    dst3 = dst.reshape(e // ce, 1, ce)
    msg = pl.pallas_call(
        functools.partial(_gather_body, ce=ce),
        grid=(e // ce,),
        in_specs=[
            pl.BlockSpec((1, 1, ce), lambda c: (c, 0, 0),
                         memory_space=pltpu.SMEM),
            pl.BlockSpec((n, f), lambda c: (0, 0)),
        ],
        out_specs=pl.BlockSpec((ce, f), lambda c: (c, 0)),
        out_shape=jax.ShapeDtypeStruct((e, f), jnp.float32),
    )(src3, h)
    return pl.pallas_call(
        functools.partial(_scatter_body, ce=ce),
        grid=(e // ce,),
        in_specs=[
            pl.BlockSpec((1, 1, ce), lambda c: (c, 0, 0),
                         memory_space=pltpu.SMEM),
            pl.BlockSpec((ce, f), lambda c: (c, 0)),
        ],
        out_specs=pl.BlockSpec((n, f), lambda c: (0, 0)),
        out_shape=jax.ShapeDtypeStruct((n, f), jnp.float32),
    )(dst3, msg)


def _scatter_rows(src, dst, h):
    n, f = h.shape
    halves = [_scatter_half(src, dst, h[:, q * 128:(q + 1) * 128])
              for q in range(f // 128)]
    return jnp.concatenate(halves, axis=1)


def _emb_body(dst_ref, et_ref, rel_ref, o_ref, *, ce):
    def body(e, _):
        j = dst_ref[0, 0, e]
        t = et_ref[0, 0, j]
        o_ref[pl.ds(e, 1), :] = rel_ref[pl.ds(t, 1), :]
        return 0

    jax.lax.fori_loop(0, ce, body, 0)


def _emb_gather(dst, edge_type, rel_table):
    e = dst.shape[0]
    nrel, hid = rel_table.shape
    ce = _pick(e, [4000, 2000, 1000, 500, 250, 125, 100, 50, 25, 10, 5, 2])
    return pl.pallas_call(
        functools.partial(_emb_body, ce=ce),
        grid=(e // ce,),
        in_specs=[
            pl.BlockSpec((1, 1, ce), lambda c: (c, 0, 0),
                         memory_space=pltpu.SMEM),
            pl.BlockSpec((1, 1, e), lambda c: (0, 0, 0),
                         memory_space=pltpu.SMEM),
            pl.BlockSpec((nrel, hid), lambda c: (0, 0)),
        ],
        out_specs=pl.BlockSpec((ce, hid), lambda c: (c, 0)),
        out_shape=jax.ShapeDtypeStruct((e, hid), jnp.float32),
    )(dst.reshape(e // ce, 1, ce), edge_type.reshape(1, 1, e), rel_table)


def kernel(x, edge_index, edge_type, rel_table, W1, b1, W2, b2,
           Wc1, bc1, Wc2, bc2):
    n = x.shape[0]
    src0 = edge_index[0]
    dst0 = edge_index[1]

    deg = _degree(dst0, n) + 1.0
    dinv = jax.lax.rsqrt(deg)
    dcol = dinv[:, None]

    hs1 = _matmul(x, W1) * dcol
    h1 = dcol * (_scatter_rows(src0, dst0, hs1) + hs1) + b1
    h1 = jnp.maximum(h1, 0.0)

    h1 = h1 + _emb_gather(dst0, edge_type, rel_table)

    hs2 = _matmul(h1, W2) * dcol
    h2 = dcol * (_scatter_rows(src0, dst0, hs2) + hs2) + b2

    h3 = _matmul(h2, Wc1, bc1, relu=True)
    ncls = Wc2.shape[1]
    wc2p = jnp.pad(Wc2, ((0, 0), (0, 128 - ncls)))
    out = _matmul(h3, wc2p)[:, :ncls] + bc2
    return out


# speedup vs baseline: 1.3664x; 1.3664x over previous
import functools
import jax
import jax.numpy as jnp
from jax.experimental import pallas as pl
from jax.experimental.pallas import tpu as pltpu


def _pick(n, cands):
    for c in cands:
        if n % c == 0:
            return c
    return 1


def _mm_body(x_ref, w_ref, b_ref, o_ref, *, relu):
    acc = jnp.dot(x_ref[...], w_ref[...], preferred_element_type=jnp.float32)
    acc = acc + b_ref[...]
    if relu:
        acc = jnp.maximum(acc, 0.0)
    o_ref[...] = acc


def _matmul(x, w, b=None, relu=False):
    m, k = x.shape
    k2, n = w.shape
    if b is None:
        b = jnp.zeros((n,), jnp.float32)
    mb = _pick(m, [1000, 500, 250, 200, 125, 100, 8, 4, 2])
    grid = (m // mb,)
    return pl.pallas_call(
        functools.partial(_mm_body, relu=relu),
        grid=grid,
        in_specs=[
            pl.BlockSpec((mb, k), lambda i: (i, 0)),
            pl.BlockSpec((k2, n), lambda i: (0, 0)),
            pl.BlockSpec((1, n), lambda i: (0, 0)),
        ],
        out_specs=pl.BlockSpec((mb, n), lambda i: (i, 0)),
        out_shape=jax.ShapeDtypeStruct((m, n), jnp.float32),
    )(x, w, b.reshape(1, n))


def _hist_body(dst_ref, o_ref, *, ce):
    c = pl.program_id(0)

    @pl.when(c == 0)
    def _():
        o_ref[...] = jnp.zeros_like(o_ref)

    lanes = jax.lax.broadcasted_iota(jnp.int32, (1, 128), 1)

    def body(e, _):
        d = dst_ref[0, 0, e]
        r = d // 128
        l = d % 128
        o_ref[pl.ds(r, 1), :] += (lanes == l).astype(jnp.float32)
        return 0

    jax.lax.fori_loop(0, ce, body, 0)


def _degree(dst, n):
    e = dst.shape[0]
    ce = _pick(e, [4000, 2000, 1000, 500, 250, 125, 100, 50, 25, 10, 5, 2])
    nr = (n + 127) // 128
    nrp = ((nr + 7) // 8) * 8
    out = pl.pallas_call(
        functools.partial(_hist_body, ce=ce),
        grid=(e // ce,),
        in_specs=[
            pl.BlockSpec((1, 1, ce), lambda c: (c, 0, 0),
                         memory_space=pltpu.SMEM),
        ],
        out_specs=pl.BlockSpec((nrp, 128), lambda c: (0, 0)),
        out_shape=jax.ShapeDtypeStruct((nrp, 128), jnp.float32),
    )(dst.reshape(e // ce, 1, ce))
    return out.reshape(-1)[:n]


def _gather_body(idx_ref, h_ref, o_ref, *, ce):
    u = 4 if ce % 4 == 0 else 1

    def body(i, _):
        e = i * u
        for q in range(u):
            s = idx_ref[0, 0, e + q]
            o_ref[pl.ds(e + q, 1), :] = h_ref[pl.ds(s, 1), :]
        return 0

    jax.lax.fori_loop(0, ce // u, body, 0)


def _scatter_body(dst_ref, m_ref, o_ref, *, ce):
    c = pl.program_id(0)

    @pl.when(c == 0)
    def _():
        o_ref[...] = jnp.zeros_like(o_ref)

    def body(e, _):
        d = dst_ref[0, 0, e]
        o_ref[pl.ds(d, 1), :] += m_ref[pl.ds(e, 1), :]
        return 0

    jax.lax.fori_loop(0, ce, body, 0)


def _scatter_half(src, dst, h):
    n, f = h.shape
    e = src.shape[0]
    ce = _pick(e, [4000, 2000, 1000, 500, 250, 125, 100, 50, 25, 10, 5, 2])
    src3 = src.reshape(e // ce, 1, ce)
    dst3 = dst.reshape(e // ce, 1, ce)
    msg = pl.pallas_call(
        functools.partial(_gather_body, ce=ce),
        grid=(e // ce,),
        in_specs=[
            pl.BlockSpec((1, 1, ce), lambda c: (c, 0, 0),
                         memory_space=pltpu.SMEM),
            pl.BlockSpec((n, f), lambda c: (0, 0)),
        ],
        out_specs=pl.BlockSpec((ce, f), lambda c: (c, 0)),
        out_shape=jax.ShapeDtypeStruct((e, f), jnp.float32),
    )(src3, h)
    return pl.pallas_call(
        functools.partial(_scatter_body, ce=ce),
        grid=(e // ce,),
        in_specs=[
            pl.BlockSpec((1, 1, ce), lambda c: (c, 0, 0),
                         memory_space=pltpu.SMEM),
            pl.BlockSpec((ce, f), lambda c: (c, 0)),
        ],
        out_specs=pl.BlockSpec((n, f), lambda c: (0, 0)),
        out_shape=jax.ShapeDtypeStruct((n, f), jnp.float32),
    )(dst3, msg)


def _scatter_rows(src, dst, h):
    n, f = h.shape
    halves = [_scatter_half(src, dst, h[:, q * 128:(q + 1) * 128])
              for q in range(f // 128)]
    return jnp.concatenate(halves, axis=1)


def _emb_body(dst_ref, et_ref, rel_ref, o_ref, *, ce):
    u = 4 if ce % 4 == 0 else 1

    def body(i, _):
        e = i * u
        for q in range(u):
            j = dst_ref[0, 0, e + q]
            t = et_ref[0, 0, j]
            o_ref[pl.ds(e + q, 1), :] = rel_ref[pl.ds(t, 1), :]
        return 0

    jax.lax.fori_loop(0, ce // u, body, 0)


def _emb_gather(dst, edge_type, rel_table):
    e = dst.shape[0]
    nrel, hid = rel_table.shape
    ce = _pick(e, [4000, 2000, 1000, 500, 250, 125, 100, 50, 25, 10, 5, 2])
    return pl.pallas_call(
        functools.partial(_emb_body, ce=ce),
        grid=(e // ce,),
        in_specs=[
            pl.BlockSpec((1, 1, ce), lambda c: (c, 0, 0),
                         memory_space=pltpu.SMEM),
            pl.BlockSpec((1, 1, e), lambda c: (0, 0, 0),
                         memory_space=pltpu.SMEM),
            pl.BlockSpec((nrel, hid), lambda c: (0, 0)),
        ],
        out_specs=pl.BlockSpec((ce, hid), lambda c: (c, 0)),
        out_shape=jax.ShapeDtypeStruct((e, hid), jnp.float32),
    )(dst.reshape(e // ce, 1, ce), edge_type.reshape(1, 1, e), rel_table)


def kernel(x, edge_index, edge_type, rel_table, W1, b1, W2, b2,
           Wc1, bc1, Wc2, bc2):
    n = x.shape[0]
    src0 = edge_index[0]
    dst0 = edge_index[1]

    deg = _degree(dst0, n) + 1.0
    dinv = jax.lax.rsqrt(deg)
    dcol = dinv[:, None]

    hs1 = _matmul(x, W1) * dcol
    h1 = dcol * (_scatter_rows(src0, dst0, hs1) + hs1) + b1
    h1 = jnp.maximum(h1, 0.0)

    h1 = h1 + _emb_gather(dst0, edge_type, rel_table)

    hs2 = _matmul(h1, W2) * dcol
    h2 = dcol * (_scatter_rows(src0, dst0, hs2) + hs2) + b2

    h3 = _matmul(h2, Wc1, bc1, relu=True)
    ncls = Wc2.shape[1]
    wc2p = jnp.pad(Wc2, ((0, 0), (0, 128 - ncls)))
    out = _matmul(h3, wc2p)[:, :ncls] + bc2
    return out


# speedup vs baseline: 1.8418x; 1.3479x over previous
import functools
import jax
import jax.numpy as jnp
from jax.experimental import pallas as pl
from jax.experimental.pallas import tpu as pltpu


def _pick(n, cands):
    for c in cands:
        if n % c == 0:
            return c
    return 1


def _mm_body(x_ref, w_ref, b_ref, o_ref, *, relu):
    acc = jnp.dot(x_ref[...], w_ref[...], preferred_element_type=jnp.float32)
    acc = acc + b_ref[...]
    if relu:
        acc = jnp.maximum(acc, 0.0)
    o_ref[...] = acc


def _matmul(x, w, b=None, relu=False):
    m, k = x.shape
    k2, n = w.shape
    if b is None:
        b = jnp.zeros((n,), jnp.float32)
    mb = _pick(m, [1000, 500, 250, 200, 125, 100, 8, 4, 2])
    grid = (m // mb,)
    return pl.pallas_call(
        functools.partial(_mm_body, relu=relu),
        grid=grid,
        in_specs=[
            pl.BlockSpec((mb, k), lambda i: (i, 0)),
            pl.BlockSpec((k2, n), lambda i: (0, 0)),
            pl.BlockSpec((1, n), lambda i: (0, 0)),
        ],
        out_specs=pl.BlockSpec((mb, n), lambda i: (i, 0)),
        out_shape=jax.ShapeDtypeStruct((m, n), jnp.float32),
    )(x, w, b.reshape(1, n))


def _hist_body(dst_ref, o_ref, *, ce):
    c = pl.program_id(0)

    @pl.when(c == 0)
    def _():
        o_ref[...] = jnp.zeros_like(o_ref)

    lanes = jax.lax.broadcasted_iota(jnp.int32, (1, 128), 1)
    u = 4 if ce % 4 == 0 else 1

    def body(i, _):
        e = i * u
        for q in range(u):
            d = dst_ref[0, 0, e + q]
            r = d // 128
            l = d % 128
            o_ref[pl.ds(r, 1), :] += (lanes == l).astype(jnp.float32)
        return 0

    jax.lax.fori_loop(0, ce // u, body, 0)


def _degree(dst, n):
    e = dst.shape[0]
    ce = _pick(e, [4000, 2000, 1000, 500, 250, 125, 100, 50, 25, 10, 5, 2])
    nr = (n + 127) // 128
    nrp = ((nr + 7) // 8) * 8
    out = pl.pallas_call(
        functools.partial(_hist_body, ce=ce),
        grid=(e // ce,),
        in_specs=[
            pl.BlockSpec((1, 1, ce), lambda c: (c, 0, 0),
                         memory_space=pltpu.SMEM),
        ],
        out_specs=pl.BlockSpec((nrp, 128), lambda c: (0, 0)),
        out_shape=jax.ShapeDtypeStruct((nrp, 128), jnp.float32),
    )(dst.reshape(e // ce, 1, ce))
    return out.reshape(-1)[:n]


def _gather_body(idx_ref, h_ref, o_ref, *, ce):
    u = 4 if ce % 4 == 0 else 1

    def body(i, _):
        e = i * u
        for q in range(u):
            s = idx_ref[0, 0, e + q]
            o_ref[pl.ds(e + q, 1), :] = h_ref[pl.ds(s, 1), :]
        return 0

    jax.lax.fori_loop(0, ce // u, body, 0)


def _scatter_body(dst_ref, m_ref, o_ref, *, ce):
    c = pl.program_id(0)

    @pl.when(c == 0)
    def _():
        o_ref[...] = jnp.zeros_like(o_ref)

    u = 4 if ce % 4 == 0 else 1

    def body(i, _):
        e = i * u
        for q in range(u):
            d = dst_ref[0, 0, e + q]
            o_ref[pl.ds(d, 1), :] += m_ref[pl.ds(e + q, 1), :]
        return 0

    jax.lax.fori_loop(0, ce // u, body, 0)


def _scatter_half(src, dst, h):
    n, f = h.shape
    e = src.shape[0]
    ce = _pick(e, [4000, 2000, 1000, 500, 250, 125, 100, 50, 25, 10, 5, 2])
    src3 = src.reshape(e // ce, 1, ce)
    dst3 = dst.reshape(e // ce, 1, ce)
    msg = pl.pallas_call(
        functools.partial(_gather_body, ce=ce),
        grid=(e // ce,),
        in_specs=[
            pl.BlockSpec((1, 1, ce), lambda c: (c, 0, 0),
                         memory_space=pltpu.SMEM),
            pl.BlockSpec((n, f), lambda c: (0, 0)),
        ],
        out_specs=pl.BlockSpec((ce, f), lambda c: (c, 0)),
        out_shape=jax.ShapeDtypeStruct((e, f), jnp.float32),
    )(src3, h)
    return pl.pallas_call(
        functools.partial(_scatter_body, ce=ce),
        grid=(e // ce,),
        in_specs=[
            pl.BlockSpec((1, 1, ce), lambda c: (c, 0, 0),
                         memory_space=pltpu.SMEM),
            pl.BlockSpec((ce, f), lambda c: (c, 0)),
        ],
        out_specs=pl.BlockSpec((n, f), lambda c: (0, 0)),
        out_shape=jax.ShapeDtypeStruct((n, f), jnp.float32),
    )(dst3, msg)


def _scatter_rows(src, dst, h):
    n, f = h.shape
    halves = [_scatter_half(src, dst, h[:, q * 128:(q + 1) * 128])
              for q in range(f // 128)]
    return jnp.concatenate(halves, axis=1)


def _emb_body(dst_ref, et_ref, rel_ref, o_ref, *, ce):
    u = 4 if ce % 4 == 0 else 1

    def body(i, _):
        e = i * u
        for q in range(u):
            j = dst_ref[0, 0, e + q]
            t = et_ref[0, 0, j]
            o_ref[pl.ds(e + q, 1), :] = rel_ref[pl.ds(t, 1), :]
        return 0

    jax.lax.fori_loop(0, ce // u, body, 0)


def _emb_gather(dst, edge_type, rel_table):
    e = dst.shape[0]
    nrel, hid = rel_table.shape
    ce = _pick(e, [4000, 2000, 1000, 500, 250, 125, 100, 50, 25, 10, 5, 2])
    return pl.pallas_call(
        functools.partial(_emb_body, ce=ce),
        grid=(e // ce,),
        in_specs=[
            pl.BlockSpec((1, 1, ce), lambda c: (c, 0, 0),
                         memory_space=pltpu.SMEM),
            pl.BlockSpec((1, 1, e), lambda c: (0, 0, 0),
                         memory_space=pltpu.SMEM),
            pl.BlockSpec((nrel, hid), lambda c: (0, 0)),
        ],
        out_specs=pl.BlockSpec((ce, hid), lambda c: (c, 0)),
        out_shape=jax.ShapeDtypeStruct((e, hid), jnp.float32),
    )(dst.reshape(e // ce, 1, ce), edge_type.reshape(1, 1, e), rel_table)


def kernel(x, edge_index, edge_type, rel_table, W1, b1, W2, b2,
           Wc1, bc1, Wc2, bc2):
    n = x.shape[0]
    src0 = edge_index[0]
    dst0 = edge_index[1]

    deg = _degree(dst0, n) + 1.0
    dinv = jax.lax.rsqrt(deg)
    dcol = dinv[:, None]

    hs1 = _matmul(x, W1) * dcol
    h1 = dcol * (_scatter_rows(src0, dst0, hs1) + hs1) + b1
    h1 = jnp.maximum(h1, 0.0)

    h1 = h1 + _emb_gather(dst0, edge_type, rel_table)

    hs2 = _matmul(h1, W2) * dcol
    h2 = dcol * (_scatter_rows(src0, dst0, hs2) + hs2) + b2

    h3 = _matmul(h2, Wc1, bc1, relu=True)
    ncls = Wc2.shape[1]
    wc2p = jnp.pad(Wc2, ((0, 0), (0, 128 - ncls)))
    out = _matmul(h3, wc2p)[:, :ncls] + bc2
    return out


# speedup vs baseline: 2.2321x; 1.2119x over previous
import functools
import jax
import jax.numpy as jnp
from jax.experimental import pallas as pl
from jax.experimental.pallas import tpu as pltpu


def _pick(n, cands):
    for c in cands:
        if n % c == 0:
            return c
    return 1


def _mm_body(x_ref, w_ref, b_ref, o_ref, *, relu):
    acc = jnp.dot(x_ref[...], w_ref[...], preferred_element_type=jnp.float32)
    acc = acc + b_ref[...]
    if relu:
        acc = jnp.maximum(acc, 0.0)
    o_ref[...] = acc


def _matmul(x, w, b=None, relu=False):
    m, k = x.shape
    k2, n = w.shape
    if b is None:
        b = jnp.zeros((n,), jnp.float32)
    mb = _pick(m, [1000, 500, 250, 200, 125, 100, 8, 4, 2])
    grid = (m // mb,)
    return pl.pallas_call(
        functools.partial(_mm_body, relu=relu),
        grid=grid,
        in_specs=[
            pl.BlockSpec((mb, k), lambda i: (i, 0)),
            pl.BlockSpec((k2, n), lambda i: (0, 0)),
            pl.BlockSpec((1, n), lambda i: (0, 0)),
        ],
        out_specs=pl.BlockSpec((mb, n), lambda i: (i, 0)),
        out_shape=jax.ShapeDtypeStruct((m, n), jnp.float32),
    )(x, w, b.reshape(1, n))


def _hist_body(dst_ref, o_ref, *, ce):
    c = pl.program_id(0)

    @pl.when(c == 0)
    def _():
        o_ref[...] = jnp.zeros_like(o_ref)

    lanes = jax.lax.broadcasted_iota(jnp.int32, (1, 128), 1)
    u = 8 if ce % 8 == 0 else (4 if ce % 4 == 0 else 1)

    def body(i, _):
        e = i * u
        for q in range(u):
            d = dst_ref[0, 0, e + q]
            r = d // 128
            l = d % 128
            o_ref[pl.ds(r, 1), :] += (lanes == l).astype(jnp.float32)
        return 0

    jax.lax.fori_loop(0, ce // u, body, 0)


def _degree(dst, n):
    e = dst.shape[0]
    ce = _pick(e, [4000, 2000, 1000, 500, 250, 125, 100, 50, 25, 10, 5, 2])
    nr = (n + 127) // 128
    nrp = ((nr + 7) // 8) * 8
    out = pl.pallas_call(
        functools.partial(_hist_body, ce=ce),
        grid=(e // ce,),
        in_specs=[
            pl.BlockSpec((1, 1, ce), lambda c: (c, 0, 0),
                         memory_space=pltpu.SMEM),
        ],
        out_specs=pl.BlockSpec((nrp, 128), lambda c: (0, 0)),
        out_shape=jax.ShapeDtypeStruct((nrp, 128), jnp.float32),
    )(dst.reshape(e // ce, 1, ce))
    return out.reshape(-1)[:n]


def _gather_body(idx_ref, h_ref, o_ref, *, ce):
    u = 8 if ce % 8 == 0 else (4 if ce % 4 == 0 else 1)

    def body(i, _):
        e = i * u
        for q in range(u):
            s = idx_ref[0, 0, e + q]
            o_ref[pl.ds(e + q, 1), :] = h_ref[pl.ds(s, 1), :]
        return 0

    jax.lax.fori_loop(0, ce // u, body, 0)


def _scatter_body(dst_ref, m_ref, o_ref, *, ce):
    c = pl.program_id(0)

    @pl.when(c == 0)
    def _():
        o_ref[...] = jnp.zeros_like(o_ref)

    u = 8 if ce % 8 == 0 else (4 if ce % 4 == 0 else 1)

    def body(i, _):
        e = i * u
        for q in range(u):
            d = dst_ref[0, 0, e + q]
            o_ref[pl.ds(d, 1), :] += m_ref[pl.ds(e + q, 1), :]
        return 0

    jax.lax.fori_loop(0, ce // u, body, 0)


def _scatter_half(src, dst, h):
    n, f = h.shape
    e = src.shape[0]
    ce = _pick(e, [4000, 2000, 1000, 500, 250, 125, 100, 50, 25, 10, 5, 2])
    src3 = src.reshape(e // ce, 1, ce)
    dst3 = dst.reshape(e // ce, 1, ce)
    msg = pl.pallas_call(
        functools.partial(_gather_body, ce=ce),
        grid=(e // ce,),
        in_specs=[
            pl.BlockSpec((1, 1, ce), lambda c: (c, 0, 0),
                         memory_space=pltpu.SMEM),
            pl.BlockSpec((n, f), lambda c: (0, 0)),
        ],
        out_specs=pl.BlockSpec((ce, f), lambda c: (c, 0)),
        out_shape=jax.ShapeDtypeStruct((e, f), jnp.float32),
    )(src3, h)
    return pl.pallas_call(
        functools.partial(_scatter_body, ce=ce),
        grid=(e // ce,),
        in_specs=[
            pl.BlockSpec((1, 1, ce), lambda c: (c, 0, 0),
                         memory_space=pltpu.SMEM),
            pl.BlockSpec((ce, f), lambda c: (c, 0)),
        ],
        out_specs=pl.BlockSpec((n, f), lambda c: (0, 0)),
        out_shape=jax.ShapeDtypeStruct((n, f), jnp.float32),
    )(dst3, msg)


def _scatter_rows(src, dst, h):
    n, f = h.shape
    halves = [_scatter_half(src, dst, h[:, q * 128:(q + 1) * 128])
              for q in range(f // 128)]
    return jnp.concatenate(halves, axis=1)


def _emb_body(dst_ref, et_ref, rel_ref, o_ref, *, ce):
    u = 8 if ce % 8 == 0 else (4 if ce % 4 == 0 else 1)

    def body(i, _):
        e = i * u
        for q in range(u):
            j = dst_ref[0, 0, e + q]
            t = et_ref[0, 0, j]
            o_ref[pl.ds(e + q, 1), :] = rel_ref[pl.ds(t, 1), :]
        return 0

    jax.lax.fori_loop(0, ce // u, body, 0)


def _emb_gather(dst, edge_type, rel_table):
    e = dst.shape[0]
    nrel, hid = rel_table.shape
    ce = _pick(e, [4000, 2000, 1000, 500, 250, 125, 100, 50, 25, 10, 5, 2])
    return pl.pallas_call(
        functools.partial(_emb_body, ce=ce),
        grid=(e // ce,),
        in_specs=[
            pl.BlockSpec((1, 1, ce), lambda c: (c, 0, 0),
                         memory_space=pltpu.SMEM),
            pl.BlockSpec((1, 1, e), lambda c: (0, 0, 0),
                         memory_space=pltpu.SMEM),
            pl.BlockSpec((nrel, hid), lambda c: (0, 0)),
        ],
        out_specs=pl.BlockSpec((ce, hid), lambda c: (c, 0)),
        out_shape=jax.ShapeDtypeStruct((e, hid), jnp.float32),
    )(dst.reshape(e // ce, 1, ce), edge_type.reshape(1, 1, e), rel_table)


def kernel(x, edge_index, edge_type, rel_table, W1, b1, W2, b2,
           Wc1, bc1, Wc2, bc2):
    n = x.shape[0]
    src0 = edge_index[0]
    dst0 = edge_index[1]

    deg = _degree(dst0, n) + 1.0
    dinv = jax.lax.rsqrt(deg)
    dcol = dinv[:, None]

    hs1 = _matmul(x, W1) * dcol
    h1 = dcol * (_scatter_rows(src0, dst0, hs1) + hs1) + b1
    h1 = jnp.maximum(h1, 0.0)

    h1 = h1 + _emb_gather(dst0, edge_type, rel_table)

    hs2 = _matmul(h1, W2) * dcol
    h2 = dcol * (_scatter_rows(src0, dst0, hs2) + hs2) + b2

    h3 = _matmul(h2, Wc1, bc1, relu=True)
    ncls = Wc2.shape[1]
    wc2p = jnp.pad(Wc2, ((0, 0), (0, 128 - ncls)))
    out = _matmul(h3, wc2p)[:, :ncls] + bc2
    return out
